# Initial kernel scaffold; baseline (speedup 1.0000x reference)
import jax
import jax.numpy as jnp
from jax.experimental import pallas as pl


def kernel(edge_index, edge_data, node_data, node_num, edge_num, padded_index, padding_mask, padded_node_mask, padded_edge_mask, emb_table, order_table):
    raise NotImplementedError("write your pallas kernel here")


# speedup vs baseline: 215.5136x; 215.5136x over previous
import functools

import jax
import jax.numpy as jnp
from jax import lax
from jax.experimental import pallas as pl
from jax.experimental.pallas import tpu as pltpu
from jax.experimental.pallas import tpu_sc as plsc

L = 16


def _tokenizer_sc(idx_flat, pa, pb, emb_flat, ord_flat, *, n_workers,
                  rows_per_tile, chunk_rows):
    n_rows = pa.shape[0]
    d = 64
    chunk_elts = chunk_rows * d
    n_chunks = rows_per_tile // chunk_rows
    vocab = emb_flat.shape[0]

    mesh = plsc.VectorSubcoreMesh(core_axis_name="c", subcore_axis_name="s")

    @functools.partial(
        pl.kernel,
        mesh=mesh,
        out_type=jax.ShapeDtypeStruct((n_rows * d,), jnp.float32),
        compiler_params=pltpu.CompilerParams(needs_layout_passes=False),
        scratch_types=[
            pltpu.VMEM((vocab,), jnp.float32),
            pltpu.VMEM((2 * d,), jnp.float32),
            pltpu.VMEM((chunk_elts,), jnp.int32),
            pltpu.VMEM((chunk_elts,), jnp.float32),
            pltpu.VMEM((chunk_rows,), jnp.int32),
            pltpu.VMEM((chunk_rows,), jnp.int32),
            pltpu.VMEM((chunk_rows,), jnp.int32),
        ],
    )
    def k(idx_hbm, pa_hbm, pb_hbm, emb_hbm, ord_hbm, out_hbm,
          emb_v, ord_v, idx_v, out_v, pa_v, pb_v, ob_v):
        wid = lax.axis_index("s") * 2 + lax.axis_index("c")
        pltpu.sync_copy(emb_hbm, emb_v)
        pltpu.sync_copy(ord_hbm, ord_v)
        row_base = wid * rows_per_tile
        iota = lax.iota(jnp.int32, L)

        def chunk_body(c, _):
            row0 = row_base + c * chunk_rows
            e0 = row0 * d
            pltpu.sync_copy(idx_hbm.at[pl.ds(e0, chunk_elts)], idx_v)
            pltpu.sync_copy(pa_hbm.at[pl.ds(row0, chunk_rows)], pa_v)
            pltpu.sync_copy(pb_hbm.at[pl.ds(row0, chunk_rows)], pb_v)

            def ob_body(i, _):
                s = pl.ds(i * L, L)
                eq = pa_v[s] == pb_v[s]
                ob_v[s] = jnp.where(eq, d, 0)
                return 0

            lax.fori_loop(0, chunk_rows // L, ob_body, 0)

            def row_body(r, _):
                ovec = plsc.load_gather(ob_v, [jnp.full((L,), r, jnp.int32)])
                base = r * d
                for kk in range(d // L):
                    s = pl.ds(base + kk * L, L)
                    vi = idx_v[s]
                    ev = plsc.load_gather(emb_v, [vi])
                    ov = plsc.load_gather(ord_v, [ovec + (kk * L + iota)])
                    out_v[s] = ev + ov
                return 0

            lax.fori_loop(0, chunk_rows, row_body, 0)
            pltpu.sync_copy(out_v, out_hbm.at[pl.ds(e0, chunk_elts)])
            return 0

        lax.fori_loop(0, n_chunks, chunk_body, 0)

    return k(idx_flat, pa, pb, emb_flat, ord_flat)


def kernel(edge_index, edge_data, node_data, node_num, edge_num, padded_index,
           padding_mask, padded_node_mask, padded_edge_mask, emb_table,
           order_table):
    b, t = padded_node_mask.shape
    d = node_data.shape[-1]
    n_rows = b * t

    n_workers = 32
    rows_per_tile = n_rows // n_workers
    chunk_rows = rows_per_tile
    for cand in (144, 171, 128, 112, 96, 72, 57, 48, 16, 8):
        if rows_per_tile % cand == 0:
            chunk_rows = cand
            break

    idx_flat = node_data.reshape(-1)
    pa = padded_index[:, :, 0].reshape(-1)
    pb = padded_index[:, :, 1].reshape(-1)
    emb_flat = emb_table.reshape(-1)
    ord_flat = order_table.reshape(-1)

    out = _tokenizer_sc(idx_flat, pa, pb, emb_flat, ord_flat,
                        n_workers=n_workers, rows_per_tile=rows_per_tile,
                        chunk_rows=chunk_rows)
    return out.reshape(b, t, d)


# speedup vs baseline: 267.6084x; 1.2417x over previous
import functools

import jax
import jax.numpy as jnp
from jax import lax
from jax.experimental import pallas as pl
from jax.experimental.pallas import tpu as pltpu
from jax.experimental.pallas import tpu_sc as plsc

L = 16
N_WORKERS = 32


def _tokenizer_sc(idx_flat, pa, pb, emb_flat, ord_flat, *, d, rows_per_tile,
                  chunk_rows):
    n_rows = pa.shape[0]
    chunk_elts = chunk_rows * d
    n_chunks = rows_per_tile // chunk_rows
    vocab = emb_flat.shape[0]
    kparts = d // L

    mesh = plsc.VectorSubcoreMesh(core_axis_name="c", subcore_axis_name="s")

    @functools.partial(
        pl.kernel,
        mesh=mesh,
        out_type=jax.ShapeDtypeStruct((n_rows * d,), jnp.float32),
        compiler_params=pltpu.CompilerParams(needs_layout_passes=False),
        scratch_types=[
            pltpu.VMEM((vocab,), jnp.float32),
            pltpu.VMEM((2 * d,), jnp.float32),
            pltpu.VMEM((rows_per_tile,), jnp.float32),
            pltpu.VMEM((rows_per_tile,), jnp.int32),
            pltpu.VMEM((rows_per_tile,), jnp.int32),
            pltpu.VMEM((chunk_elts,), jnp.int32),
            pltpu.VMEM((chunk_elts,), jnp.int32),
            pltpu.VMEM((chunk_elts,), jnp.float32),
            pltpu.VMEM((chunk_elts,), jnp.float32),
            pltpu.SemaphoreType.DMA,
            pltpu.SemaphoreType.DMA,
            pltpu.SemaphoreType.DMA,
            pltpu.SemaphoreType.DMA,
        ],
    )
    def k(idx_hbm, pa_hbm, pb_hbm, emb_hbm, ord_hbm, out_hbm,
          emb_v, ord_v, ob_v, pa_v, pb_v, idx0, idx1, o0, o1,
          si0, si1, so0, so1):
        wid = lax.axis_index("s") * 2 + lax.axis_index("c")
        row_base = wid * rows_per_tile
        ebase = row_base * d
        pltpu.sync_copy(emb_hbm, emb_v)
        pltpu.sync_copy(ord_hbm, ord_v)
        pltpu.sync_copy(pa_hbm.at[pl.ds(row_base, rows_per_tile)], pa_v)
        pltpu.sync_copy(pb_hbm.at[pl.ds(row_base, rows_per_tile)], pb_v)

        def ob_body(i, _):
            s = pl.ds(i * L, L)
            ob_v[s] = jnp.where(pa_v[s] == pb_v[s], 1.0, 0.0)
            return 0

        lax.fori_loop(0, rows_per_tile // L, ob_body, 0)

        ord0 = [ord_v[pl.ds(kk * L, L)] for kk in range(kparts)]
        dord = [ord_v[pl.ds(d + kk * L, L)] - ord0[kk] for kk in range(kparts)]

        idx_bufs = (idx0, idx1)
        out_bufs = (o0, o1)
        sin = (si0, si1)
        sout = (so0, so1)

        def in_copy(c, b):
            return pltpu.make_async_copy(
                idx_hbm.at[pl.ds(ebase + c * chunk_elts, chunk_elts)],
                idx_bufs[b], sin[b])

        def out_copy(c, b):
            return pltpu.make_async_copy(
                out_bufs[b],
                out_hbm.at[pl.ds(ebase + c * chunk_elts, chunk_elts)],
                sout[b])

        in_copy(0, 0).start()
        in_copy(1, 1).start()

        def compute(c, b):
            ib = idx_bufs[b]
            outb = out_bufs[b]

            def row_body(r4, _):
                for rr in range(4):
                    r = r4 * 4 + rr
                    ro = c * chunk_rows + r
                    ovf = plsc.load_gather(
                        ob_v, [jnp.full((L,), ro, jnp.int32)])
                    base = r * d
                    for kk in range(kparts):
                        s = pl.ds(base + kk * L, L)
                        ev = plsc.load_gather(emb_v, [ib[s]])
                        outb[s] = ev + (ord0[kk] + ovf * dord[kk])
                return 0

            lax.fori_loop(0, chunk_rows // 4, row_body, 0)

        def pair_body(c2, _):
            for b in range(2):
                c = c2 * 2 + b

                in_copy(c, b).wait()

                @pl.when(c2 > 0)
                def _():
                    out_copy(c - 2, b).wait()

                compute(c, b)
                out_copy(c, b).start()

                @pl.when(c2 < n_chunks // 2 - 1)
                def _():
                    in_copy(c + 2, b).start()

            return 0

        lax.fori_loop(0, n_chunks // 2, pair_body, 0)
        out_copy(n_chunks - 2, 0).wait()
        out_copy(n_chunks - 1, 1).wait()

    return k(idx_flat, pa, pb, emb_flat, ord_flat)


def kernel(edge_index, edge_data, node_data, node_num, edge_num, padded_index,
           padding_mask, padded_node_mask, padded_edge_mask, emb_table,
           order_table):
    b, t = padded_node_mask.shape
    d = node_data.shape[-1]
    n_rows = b * t

    rows_per_tile = n_rows // N_WORKERS
    chunk_rows = None
    for cand in (228, 144, 152, 108, 76, 72, 54, 36, 16, 8, 4):
        if (rows_per_tile % cand == 0 and (rows_per_tile // cand) % 2 == 0
                and cand % 4 == 0):
            chunk_rows = cand
            break
    assert chunk_rows is not None

    idx_flat = node_data.reshape(-1)
    pa = padded_index[:, :, 0].reshape(-1)
    pb = padded_index[:, :, 1].reshape(-1)
    emb_flat = emb_table.reshape(-1)
    ord_flat = order_table.reshape(-1)

    out = _tokenizer_sc(idx_flat, pa, pb, emb_flat, ord_flat, d=d,
                        rows_per_tile=rows_per_tile, chunk_rows=chunk_rows)
    return out.reshape(b, t, d)


# speedup vs baseline: 454.5303x; 1.6985x over previous
import functools

import jax
import jax.numpy as jnp
from jax import lax
from jax.experimental import pallas as pl
from jax.experimental.pallas import tpu as pltpu
from jax.experimental.pallas import tpu_sc as plsc

L = 16
N_WORKERS = 32


def _tokenizer_sc(idx_flat, pa, pb, emb_flat, ord_flat, *, d, rows_per_tile,
                  chunk_rows):
    n_rows = pa.shape[0]
    chunk_elts = chunk_rows * d
    n_chunks = rows_per_tile // chunk_rows
    vocab = emb_flat.shape[0]
    kparts = d // L

    mesh = plsc.VectorSubcoreMesh(core_axis_name="c", subcore_axis_name="s")

    @functools.partial(
        pl.kernel,
        mesh=mesh,
        out_type=jax.ShapeDtypeStruct((n_rows * d,), jnp.float32),
        compiler_params=pltpu.CompilerParams(needs_layout_passes=False),
        scratch_types=[
            pltpu.VMEM((vocab,), jnp.float32),
            pltpu.VMEM((2 * d,), jnp.float32),
            pltpu.VMEM((rows_per_tile,), jnp.float32),
            pltpu.VMEM((rows_per_tile,), jnp.int32),
            pltpu.VMEM((rows_per_tile,), jnp.int32),
            pltpu.VMEM((chunk_elts,), jnp.int32),
            pltpu.VMEM((chunk_elts,), jnp.int32),
            pltpu.VMEM((chunk_elts,), jnp.float32),
            pltpu.VMEM((chunk_elts,), jnp.float32),
            pltpu.SemaphoreType.DMA,
            pltpu.SemaphoreType.DMA,
            pltpu.SemaphoreType.DMA,
            pltpu.SemaphoreType.DMA,
        ],
    )
    def k(idx_hbm, pa_hbm, pb_hbm, emb_hbm, ord_hbm, out_hbm,
          emb_v, ord_v, ob_v, pa_v, pb_v, idx0, idx1, o0, o1,
          si0, si1, so0, so1):
        wid = lax.axis_index("s") * 2 + lax.axis_index("c")
        row_base = wid * rows_per_tile
        ebase = row_base * d
        pltpu.sync_copy(emb_hbm, emb_v)
        pltpu.sync_copy(ord_hbm, ord_v)
        pltpu.sync_copy(pa_hbm.at[pl.ds(row_base, rows_per_tile)], pa_v)
        pltpu.sync_copy(pb_hbm.at[pl.ds(row_base, rows_per_tile)], pb_v)

        @plsc.parallel_loop(0, rows_per_tile // L, unroll=4)
        def ob_body(i):
            s = pl.ds(i * L, L)
            ob_v[s] = jnp.where(pa_v[s] == pb_v[s], 1.0, 0.0)

        ord0 = [ord_v[pl.ds(kk * L, L)] for kk in range(kparts)]
        dord = [ord_v[pl.ds(d + kk * L, L)] - ord0[kk] for kk in range(kparts)]

        idx_bufs = (idx0, idx1)
        out_bufs = (o0, o1)
        sin = (si0, si1)
        sout = (so0, so1)

        def in_copy(c, b):
            return pltpu.make_async_copy(
                idx_hbm.at[pl.ds(ebase + c * chunk_elts, chunk_elts)],
                idx_bufs[b], sin[b])

        def out_copy(c, b):
            return pltpu.make_async_copy(
                out_bufs[b],
                out_hbm.at[pl.ds(ebase + c * chunk_elts, chunk_elts)],
                sout[b])

        in_copy(0, 0).start()
        in_copy(1, 1).start()

        def compute(c, b):
            ib = idx_bufs[b]
            outb = out_bufs[b]

            @plsc.parallel_loop(0, chunk_rows, unroll=4)
            def row_body(r):
                ro = c * chunk_rows + r
                ovf = plsc.load_gather(
                    ob_v, [jnp.full((L,), ro, jnp.int32)])
                base = r * d
                for kk in range(kparts):
                    s = pl.ds(base + kk * L, L)
                    ev = plsc.load_gather(emb_v, [ib[s]])
                    outb[s] = ev + (ord0[kk] + ovf * dord[kk])

        def pair_body(c2, _):
            for b in range(2):
                c = c2 * 2 + b

                in_copy(c, b).wait()

                @pl.when(c2 > 0)
                def _():
                    out_copy(c - 2, b).wait()

                compute(c, b)
                out_copy(c, b).start()

                @pl.when(c2 < n_chunks // 2 - 1)
                def _():
                    in_copy(c + 2, b).start()

            return 0

        lax.fori_loop(0, n_chunks // 2, pair_body, 0)
        out_copy(n_chunks - 2, 0).wait()
        out_copy(n_chunks - 1, 1).wait()

    return k(idx_flat, pa, pb, emb_flat, ord_flat)


def kernel(edge_index, edge_data, node_data, node_num, edge_num, padded_index,
           padding_mask, padded_node_mask, padded_edge_mask, emb_table,
           order_table):
    b, t = padded_node_mask.shape
    d = node_data.shape[-1]
    n_rows = b * t

    rows_per_tile = n_rows // N_WORKERS
    chunk_rows = None
    for cand in (228, 144, 152, 108, 76, 72, 54, 36, 16, 8, 4):
        if (rows_per_tile % cand == 0 and (rows_per_tile // cand) % 2 == 0
                and cand % 4 == 0):
            chunk_rows = cand
            break
    assert chunk_rows is not None

    idx_flat = node_data.reshape(-1)
    pa = padded_index[:, :, 0].reshape(-1)
    pb = padded_index[:, :, 1].reshape(-1)
    emb_flat = emb_table.reshape(-1)
    ord_flat = order_table.reshape(-1)

    out = _tokenizer_sc(idx_flat, pa, pb, emb_flat, ord_flat, d=d,
                        rows_per_tile=rows_per_tile, chunk_rows=chunk_rows)
    return out.reshape(b, t, d)
